# baseline (speedup 1.0000x reference)
import jax
import jax.numpy as jnp
from jax.experimental import pallas as pl
from jax.experimental.pallas import tpu as pltpu

_BB = 16


def _body(x0_ref, x1_ref, c_ref, o_ref):
    d = o_ref.shape[1]
    half = x0_ref.shape[-1]
    for k, xr in enumerate((x0_ref, x1_ref)):
        for bb in range(xr.shape[0]):
            x = xr[bb]
            e = jnp.exp(x - jnp.max(x, axis=0, keepdims=True))
            proj = jax.lax.dot_general(
                c_ref[...], e, (((0,), (0,)), ((), ())),
                preferred_element_type=jnp.float32,
            )
            o_ref[bb, :, k * half:(k + 1) * half] = proj[:d] * (1.0 / proj[d:d + 1])


def kernel(base_logits, compatibility_matrix):
    b, seq, vocab = base_logits.shape
    diac = compatibility_matrix.shape[1]

    xt = jnp.transpose(base_logits, (0, 2, 1))
    caug = jnp.concatenate(
        [compatibility_matrix, jnp.ones((vocab, 1), jnp.float32)], axis=1
    )
    out_t = pl.pallas_call(
        _body,
        grid=(b // _BB,),
        in_specs=[
            pl.BlockSpec((_BB, vocab, seq // 2), lambda i: (i, 0, 0)),
            pl.BlockSpec((_BB, vocab, seq // 2), lambda i: (i, 0, 1)),
            pl.BlockSpec((vocab, diac + 1), lambda i: (0, 0)),
        ],
        out_specs=pl.BlockSpec((_BB, diac, seq), lambda i: (i, 0, 0)),
        out_shape=jax.ShapeDtypeStruct((b, diac, seq), jnp.float32),
        compiler_params=pltpu.CompilerParams(
            dimension_semantics=("parallel",),
        ),
    )(xt, xt, caug)
    return jnp.transpose(out_t, (0, 2, 1))
